# baseline (speedup 1.0000x reference)
import jax
import jax.numpy as jnp
from jax.experimental import pallas as pl

_H = 64


def _layer0_body(x_ref, a_ref, wr_ref, wl_ref, b_ref, o_ref):
    acc = a_ref[...] * wl_ref[...] + x_ref[...] * wr_ref[...] + b_ref[...]
    o_ref[...] = jnp.maximum(acc, 0.0)


def _layer_body(h_ref, a_ref, wr_ref, wl_ref, b_ref, o_ref):
    acc = jnp.dot(a_ref[...], wl_ref[...], preferred_element_type=jnp.float32)
    acc += jnp.dot(h_ref[...], wr_ref[...], preferred_element_type=jnp.float32)
    acc += b_ref[...]
    o_ref[...] = jnp.maximum(acc, 0.0)


def _dense_update(h, agg, Wl, bl, Wr):
    n, din = h.shape
    blk = 2000
    grid = n // blk
    body = _layer0_body if din == 1 else _layer_body
    return pl.pallas_call(
        body,
        grid=(grid,),
        in_specs=[
            pl.BlockSpec((blk, din), lambda i: (i, 0)),
            pl.BlockSpec((blk, din), lambda i: (i, 0)),
            pl.BlockSpec((din, _H), lambda i: (0, 0)),
            pl.BlockSpec((din, _H), lambda i: (0, 0)),
            pl.BlockSpec((1, _H), lambda i: (0, 0)),
        ],
        out_specs=pl.BlockSpec((blk, _H), lambda i: (i, 0)),
        out_shape=jax.ShapeDtypeStruct((n, _H), jnp.float32),
    )(h, agg, Wr.T, Wl.T, bl[None, :])


def _combine_body(zs_ref, zd_ref, wc_ref, bc_ref, o_ref):
    ef = zs_ref[...] * zd_ref[...]
    s = jnp.dot(ef, wc_ref[...], preferred_element_type=jnp.float32)[:, 0]
    o_ref[...] = jax.nn.sigmoid(s + bc_ref[0, 0])


def _combine(z_src, z_dst, Wc, bc):
    q = z_src.shape[0]
    blk = 8192
    grid = q // blk
    return pl.pallas_call(
        _combine_body,
        grid=(grid,),
        in_specs=[
            pl.BlockSpec((blk, _H), lambda i: (i, 0)),
            pl.BlockSpec((blk, _H), lambda i: (i, 0)),
            pl.BlockSpec((_H, 1), lambda i: (0, 0)),
            pl.BlockSpec((1, 1), lambda i: (0, 0)),
        ],
        out_specs=pl.BlockSpec((blk,), lambda i: (i,)),
        out_shape=jax.ShapeDtypeStruct((q,), jnp.float32),
    )(z_src, z_dst, Wc.T, bc[None, :])


def kernel(x, edge_index, query_edges, Wl0, bl0, Wr0, Wl1, bl1, Wr1,
           Wl2, bl2, Wr2, Wl3, bl3, Wr3, Wc, bc):
    n = x.shape[0]
    src, dst = edge_index[0], edge_index[1]
    h = x
    for (Wl, bl, Wr) in ((Wl0, bl0, Wr0), (Wl1, bl1, Wr1),
                         (Wl2, bl2, Wr2), (Wl3, bl3, Wr3)):
        msgs = h[src]
        agg = jax.ops.segment_max(msgs, dst, num_segments=n)
        agg = jnp.where(jnp.isfinite(agg), agg, 0.0)
        h = _dense_update(h, agg, Wl, bl, Wr)
    q = query_edges.shape[1]
    blk = 8192
    q_pad = ((q + blk - 1) // blk) * blk
    pad = jnp.zeros((q_pad - q,), dtype=query_edges.dtype)
    qi_src = jnp.concatenate([query_edges[0], pad])
    qi_dst = jnp.concatenate([query_edges[1], pad])
    z_src = h[qi_src]
    z_dst = h[qi_dst]
    return _combine(z_src, z_dst, Wc, bc)[:q]


# speedup vs baseline: 2.1117x; 2.1117x over previous
import functools

import jax
import jax.numpy as jnp
from jax import lax
from jax.experimental import pallas as pl
from jax.experimental.pallas import tpu as pltpu
from jax.experimental.pallas import tpu_sc as plsc

_H = 64
_NW = 32
_S = 128
_K = 8
_W = _S * _K


def _sc_gather_call(nwin, d, table, idx):
    m = idx.shape[0]
    per_w = m // _NW
    out_shape = (m,) if d == 1 else (m, d)
    row_shape = (_W,) if d == 1 else (_W, d)
    if d == 1:
        table = table.reshape(-1)

    @functools.partial(
        pl.kernel,
        out_type=jax.ShapeDtypeStruct(out_shape, jnp.float32),
        mesh=plsc.VectorSubcoreMesh(core_axis_name="c", subcore_axis_name="s"),
        scratch_types=[
            pltpu.VMEM((_W,), jnp.int32),
            pltpu.VMEM(row_shape, jnp.float32),
            pltpu.SemaphoreType.DMA,
        ],
        compiler_params=pltpu.CompilerParams(use_tc_tiling_on_sc=False),
    )
    def k(table_hbm, idx_hbm, out_hbm, idx_v, rows_v, sem):
        wid = lax.axis_index("s") * 2 + lax.axis_index("c")
        base = wid * per_w

        def body(j, carry):
            off = base + j * _W
            pltpu.sync_copy(idx_hbm.at[pl.ds(off, _W)], idx_v)
            descs = []
            for b in range(_K):
                descs.append(pltpu.async_copy(
                    table_hbm.at[idx_v.at[pl.ds(b * _S, _S)]],
                    rows_v.at[pl.ds(b * _S, _S)],
                    sem))
            for dsc in descs:
                dsc.wait()
            pltpu.sync_copy(rows_v, out_hbm.at[pl.ds(off, _W)])
            return carry

        lax.fori_loop(0, nwin, body, 0)

    return k(table, idx)


def _pad_idx(idx, mult):
    m = idx.shape[0]
    m_pad = ((m + mult - 1) // mult) * mult
    if m_pad == m:
        return idx, m
    extra = jnp.arange(m_pad - m, dtype=idx.dtype) % jnp.int32(131)
    return jnp.concatenate([idx, extra]), m


def _sc_gather(table, idx):
    idx_p, _ = _pad_idx(idx, _NW * _W)
    nwin = idx_p.shape[0] // (_NW * _W)
    return _sc_gather_call(nwin, table.shape[1], table, idx_p)


def _layer0_body(x_ref, a_ref, wr_ref, wl_ref, b_ref, o_ref):
    acc = a_ref[...] * wl_ref[...] + x_ref[...] * wr_ref[...] + b_ref[...]
    o_ref[...] = jnp.maximum(acc, 0.0)


def _layer_body(h_ref, a_ref, wr_ref, wl_ref, b_ref, o_ref):
    acc = jnp.dot(a_ref[...], wl_ref[...], preferred_element_type=jnp.float32)
    acc += jnp.dot(h_ref[...], wr_ref[...], preferred_element_type=jnp.float32)
    acc += b_ref[...]
    o_ref[...] = jnp.maximum(acc, 0.0)


def _dense_update(h, agg, Wl, bl, Wr):
    n, din = h.shape
    blk = 2000
    grid = n // blk
    body = _layer0_body if din == 1 else _layer_body
    return pl.pallas_call(
        body,
        grid=(grid,),
        in_specs=[
            pl.BlockSpec((blk, din), lambda i: (i, 0)),
            pl.BlockSpec((blk, din), lambda i: (i, 0)),
            pl.BlockSpec((din, _H), lambda i: (0, 0)),
            pl.BlockSpec((din, _H), lambda i: (0, 0)),
            pl.BlockSpec((1, _H), lambda i: (0, 0)),
        ],
        out_specs=pl.BlockSpec((blk, _H), lambda i: (i, 0)),
        out_shape=jax.ShapeDtypeStruct((n, _H), jnp.float32),
    )(h, agg, Wr.T, Wl.T, bl[None, :])


def _combine_body(zs_ref, zd_ref, wc_ref, bc_ref, o_ref):
    ef = zs_ref[...] * zd_ref[...]
    s = jnp.dot(ef, wc_ref[...], preferred_element_type=jnp.float32)[:, 0]
    o_ref[...] = jax.nn.sigmoid(s + bc_ref[0, 0])


def _combine(z_src, z_dst, Wc, bc):
    q = z_src.shape[0]
    blk = 8192
    grid = q // blk
    return pl.pallas_call(
        _combine_body,
        grid=(grid,),
        in_specs=[
            pl.BlockSpec((blk, _H), lambda i: (i, 0)),
            pl.BlockSpec((blk, _H), lambda i: (i, 0)),
            pl.BlockSpec((_H, 1), lambda i: (0, 0)),
            pl.BlockSpec((1, 1), lambda i: (0, 0)),
        ],
        out_specs=pl.BlockSpec((blk,), lambda i: (i,)),
        out_shape=jax.ShapeDtypeStruct((q,), jnp.float32),
    )(z_src, z_dst, Wc.T, bc[None, :])


def kernel(x, edge_index, query_edges, Wl0, bl0, Wr0, Wl1, bl1, Wr1,
           Wl2, bl2, Wr2, Wl3, bl3, Wr3, Wc, bc):
    n = x.shape[0]
    e = edge_index.shape[1]
    src, dst = edge_index[0], edge_index[1]
    src_p, _ = _pad_idx(src, _NW * _W)
    e_pad = src_p.shape[0]
    dst_p = jnp.concatenate([dst, jnp.full((e_pad - e,), n, dtype=dst.dtype)])

    h = x
    for li, (Wl, bl, Wr) in enumerate(((Wl0, bl0, Wr0), (Wl1, bl1, Wr1),
                                       (Wl2, bl2, Wr2), (Wl3, bl3, Wr3))):
        nwin = e_pad // (_NW * _W)
        msgs = _sc_gather_call(nwin, h.shape[1], h, src_p)
        if msgs.ndim == 1:
            msgs = msgs[:, None]
        agg = jax.ops.segment_max(msgs, dst_p, num_segments=n)
        agg = jnp.where(jnp.isfinite(agg), agg, 0.0)
        h = _dense_update(h, agg, Wl, bl, Wr)

    q = query_edges.shape[1]
    qs_p, _ = _pad_idx(query_edges[0], _NW * _W)
    qd_p, _ = _pad_idx(query_edges[1], _NW * _W)
    z_src = _sc_gather(h, qs_p)
    z_dst = _sc_gather(h, qd_p)
    return _combine(z_src, z_dst, Wc, bc)[:q]


# speedup vs baseline: 2.4835x; 1.1761x over previous
import functools

import jax
import jax.numpy as jnp
from jax import lax
from jax.experimental import pallas as pl
from jax.experimental.pallas import tpu as pltpu
from jax.experimental.pallas import tpu_sc as plsc

_H = 64
_NW = 32
_S = 128
_K = 8
_W = _S * _K


def _sc_gather_call(nwin, d, table, idx):
    m = idx.shape[0]
    per_w = m // _NW
    out_shape = (m,) if d == 1 else (m, d)
    row_shape = (_W,) if d == 1 else (_W, d)
    if d == 1:
        table = table.reshape(-1)

    @functools.partial(
        pl.kernel,
        out_type=jax.ShapeDtypeStruct(out_shape, jnp.float32),
        mesh=plsc.VectorSubcoreMesh(core_axis_name="c", subcore_axis_name="s"),
        scratch_types=[
            pltpu.VMEM((_W,), jnp.int32),
            pltpu.VMEM(row_shape, jnp.float32),
            pltpu.SemaphoreType.DMA,
        ],
        compiler_params=pltpu.CompilerParams(use_tc_tiling_on_sc=False),
    )
    def k(table_hbm, idx_hbm, out_hbm, idx_v, rows_v, sem):
        wid = lax.axis_index("s") * 2 + lax.axis_index("c")
        base = wid * per_w

        def body(j, carry):
            off = base + j * _W
            pltpu.sync_copy(idx_hbm.at[pl.ds(off, _W)], idx_v)
            descs = []
            for b in range(_K):
                descs.append(pltpu.async_copy(
                    table_hbm.at[idx_v.at[pl.ds(b * _S, _S)]],
                    rows_v.at[pl.ds(b * _S, _S)],
                    sem))
            for dsc in descs:
                dsc.wait()
            pltpu.sync_copy(rows_v, out_hbm.at[pl.ds(off, _W)])
            return carry

        lax.fori_loop(0, nwin, body, 0)

    return k(table, idx)


def _pad_idx(idx, mult):
    m = idx.shape[0]
    m_pad = ((m + mult - 1) // mult) * mult
    if m_pad == m:
        return idx, m
    extra = jnp.arange(m_pad - m, dtype=idx.dtype) % jnp.int32(131)
    return jnp.concatenate([idx, extra]), m


def _sc_gather(table, idx):
    idx_p, _ = _pad_idx(idx, _NW * _W)
    nwin = idx_p.shape[0] // (_NW * _W)
    return _sc_gather_call(nwin, table.shape[1], table, idx_p)


def _layer0_body(x_ref, a_ref, wr_ref, wl_ref, b_ref, o_ref):
    acc = a_ref[...] * wl_ref[...] + x_ref[...] * wr_ref[...] + b_ref[...]
    o_ref[...] = jnp.maximum(acc, 0.0)


def _layer_body(h_ref, a_ref, wr_ref, wl_ref, b_ref, o_ref):
    acc = jnp.dot(a_ref[...], wl_ref[...], preferred_element_type=jnp.float32)
    acc += jnp.dot(h_ref[...], wr_ref[...], preferred_element_type=jnp.float32)
    acc += b_ref[...]
    o_ref[...] = jnp.maximum(acc, 0.0)


def _dense_update(h, agg, Wl, bl, Wr):
    n, din = h.shape
    blk = 2000
    grid = n // blk
    body = _layer0_body if din == 1 else _layer_body
    return pl.pallas_call(
        body,
        grid=(grid,),
        in_specs=[
            pl.BlockSpec((blk, din), lambda i: (i, 0)),
            pl.BlockSpec((blk, din), lambda i: (i, 0)),
            pl.BlockSpec((din, _H), lambda i: (0, 0)),
            pl.BlockSpec((din, _H), lambda i: (0, 0)),
            pl.BlockSpec((1, _H), lambda i: (0, 0)),
        ],
        out_specs=pl.BlockSpec((blk, _H), lambda i: (i, 0)),
        out_shape=jax.ShapeDtypeStruct((n, _H), jnp.float32),
    )(h, agg, Wr.T, Wl.T, bl[None, :])


def _combine_body(zs_ref, zd_ref, wc_ref, bc_ref, o_ref):
    ef = zs_ref[...] * zd_ref[...]
    s = jnp.dot(ef, wc_ref[...], preferred_element_type=jnp.float32)[:, 0]
    o_ref[...] = jax.nn.sigmoid(s + bc_ref[0, 0])


def _combine(z_src, z_dst, Wc, bc):
    q = z_src.shape[0]
    blk = 8192
    grid = q // blk
    return pl.pallas_call(
        _combine_body,
        grid=(grid,),
        in_specs=[
            pl.BlockSpec((blk, _H), lambda i: (i, 0)),
            pl.BlockSpec((blk, _H), lambda i: (i, 0)),
            pl.BlockSpec((_H, 1), lambda i: (0, 0)),
            pl.BlockSpec((1, 1), lambda i: (0, 0)),
        ],
        out_specs=pl.BlockSpec((blk,), lambda i: (i,)),
        out_shape=jax.ShapeDtypeStruct((q,), jnp.float32),
    )(z_src, z_dst, Wc.T, bc[None, :])


def kernel(x, edge_index, query_edges, Wl0, bl0, Wr0, Wl1, bl1, Wr1,
           Wl2, bl2, Wr2, Wl3, bl3, Wr3, Wc, bc):
    n = x.shape[0]
    e = edge_index.shape[1]
    src, dst = edge_index[0], edge_index[1]
    src_p, _ = _pad_idx(src, _NW * _W)
    e_pad = src_p.shape[0]
    dst_p = jnp.concatenate([dst, jnp.full((e_pad - e,), n, dtype=dst.dtype)])
    sdst, ssrc = lax.sort((dst_p, src_p), num_keys=1)

    h = x
    for li, (Wl, bl, Wr) in enumerate(((Wl0, bl0, Wr0), (Wl1, bl1, Wr1),
                                       (Wl2, bl2, Wr2), (Wl3, bl3, Wr3))):
        nwin = e_pad // (_NW * _W)
        msgs = _sc_gather_call(nwin, h.shape[1], h, ssrc)
        if msgs.ndim == 1:
            msgs = msgs[:, None]
        agg = jax.ops.segment_max(msgs, sdst, num_segments=n,
                                  indices_are_sorted=True)
        agg = jnp.where(jnp.isfinite(agg), agg, 0.0)
        h = _dense_update(h, agg, Wl, bl, Wr)

    q = query_edges.shape[1]
    qs_p, _ = _pad_idx(query_edges[0], _NW * _W)
    qd_p, _ = _pad_idx(query_edges[1], _NW * _W)
    z_src = _sc_gather(h, qs_p)
    z_dst = _sc_gather(h, qd_p)
    return _combine(z_src, z_dst, Wc, bc)[:q]


# speedup vs baseline: 3.3741x; 1.3586x over previous
import functools

import jax
import jax.numpy as jnp
from jax import lax
from jax.experimental import pallas as pl
from jax.experimental.pallas import tpu as pltpu
from jax.experimental.pallas import tpu_sc as plsc

_H = 64
_NW = 32
_S = 128
_K = 8
_W = _S * _K


def _sc_gather_call(nwin, d, table, idx):
    m = idx.shape[0]
    per_w = m // _NW
    out_shape = (m,) if d == 1 else (m, d)
    row_shape = (_W,) if d == 1 else (_W, d)
    if d == 1:
        table = table.reshape(-1)

    @functools.partial(
        pl.kernel,
        out_type=jax.ShapeDtypeStruct(out_shape, jnp.float32),
        mesh=plsc.VectorSubcoreMesh(core_axis_name="c", subcore_axis_name="s"),
        scratch_types=[
            pltpu.VMEM((_W,), jnp.int32),
            pltpu.VMEM(row_shape, jnp.float32),
            pltpu.SemaphoreType.DMA,
        ],
        compiler_params=pltpu.CompilerParams(use_tc_tiling_on_sc=False),
    )
    def k(table_hbm, idx_hbm, out_hbm, idx_v, rows_v, sem):
        wid = lax.axis_index("s") * 2 + lax.axis_index("c")
        base = wid * per_w

        def body(j, carry):
            off = base + j * _W
            pltpu.sync_copy(idx_hbm.at[pl.ds(off, _W)], idx_v)
            descs = []
            for b in range(_K):
                descs.append(pltpu.async_copy(
                    table_hbm.at[idx_v.at[pl.ds(b * _S, _S)]],
                    rows_v.at[pl.ds(b * _S, _S)],
                    sem))
            for dsc in descs:
                dsc.wait()
            pltpu.sync_copy(rows_v, out_hbm.at[pl.ds(off, _W)])
            return carry

        lax.fori_loop(0, nwin, body, 0)

    return k(table, idx)


def _pad_idx(idx, mult):
    m = idx.shape[0]
    m_pad = ((m + mult - 1) // mult) * mult
    if m_pad == m:
        return idx, m
    extra = jnp.arange(m_pad - m, dtype=idx.dtype) % jnp.int32(131)
    return jnp.concatenate([idx, extra]), m


def _sc_gather(table, idx):
    idx_p, _ = _pad_idx(idx, _NW * _W)
    nwin = idx_p.shape[0] // (_NW * _W)
    return _sc_gather_call(nwin, table.shape[1], table, idx_p)


_SW = 512
_NPT = 3125


def _sc_segmax(h2, sdst_p, sa, sb, offs, n):
    @functools.partial(
        pl.kernel,
        out_type=(jax.ShapeDtypeStruct((n, 32), jnp.float32),
                  jax.ShapeDtypeStruct((n, 32), jnp.float32)),
        mesh=plsc.VectorSubcoreMesh(core_axis_name="c", subcore_axis_name="s"),
        scratch_types=[
            pltpu.VMEM((64,), jnp.int32),
            pltpu.VMEM((_SW + 16,), jnp.int32),
            pltpu.VMEM((_SW,), jnp.int32),
            pltpu.VMEM((_SW, 32), jnp.float32),
            pltpu.VMEM((_NPT, 32), jnp.float32),
            pltpu.SemaphoreType.DMA,
        ],
        compiler_params=pltpu.CompilerParams(use_tc_tiling_on_sc=False),
    )
    def k(h2_hbm, dst_hbm, sa_hbm, sb_hbm, offs_hbm, lo_hbm, hi_hbm,
          offs_v, dst_v, idx_v, rows_v, acc_v, sem):
        wid = lax.axis_index("s") * 2 + lax.axis_index("c")
        pltpu.sync_copy(offs_hbm, offs_v)
        t0 = offs_v[pl.ds(wid, 16)][0]
        t1 = offs_v[pl.ds(wid + 1, 16)][0]
        base_node = wid * _NPT
        a0 = (t0 // _SW) * _SW
        nwin = (t1 - a0 + _SW - 1) // _SW
        zeros16 = jnp.zeros((16,), jnp.float32)

        for c, (src_hbm, out_hbm) in ((0, (sa_hbm, lo_hbm)),
                                      (1, (sb_hbm, hi_hbm))):
            def zbody(r, carry):
                acc_v[r, pl.ds(0, 16)] = zeros16
                acc_v[r, pl.ds(16, 16)] = zeros16
                return carry
            lax.fori_loop(0, _NPT, zbody, 0)

            def wbody(w, carry, src_hbm=src_hbm):
                start = a0 + w * _SW
                pltpu.sync_copy(dst_hbm.at[pl.ds(start, _SW)],
                                dst_v.at[pl.ds(0, _SW)])
                pltpu.sync_copy(src_hbm.at[pl.ds(start, _SW)], idx_v)
                descs = []
                for b in range(_SW // _S):
                    descs.append(pltpu.async_copy(
                        h2_hbm.at[idx_v.at[pl.ds(b * _S, _S)]],
                        rows_v.at[pl.ds(b * _S, _S)],
                        sem))
                for dsc in descs:
                    dsc.wait()
                jlo = jnp.maximum(t0 - start, 0)
                jhi = jnp.minimum(t1 - start, _SW)

                def ebody(j, carry):
                    d = dst_v[pl.ds(j, 16)][0] - base_node
                    r0 = rows_v[j, pl.ds(0, 16)]
                    r1 = rows_v[j, pl.ds(16, 16)]
                    acc_v[d, pl.ds(0, 16)] = jnp.maximum(
                        acc_v[d, pl.ds(0, 16)], r0)
                    acc_v[d, pl.ds(16, 16)] = jnp.maximum(
                        acc_v[d, pl.ds(16, 16)], r1)
                    return carry
                lax.fori_loop(jlo, jhi, ebody, 0)
                return carry
            lax.fori_loop(0, nwin, wbody, 0)
            pltpu.sync_copy(acc_v, out_hbm.at[pl.ds(base_node, _NPT)])

    return k(h2, sdst_p, sa, sb, offs)


def _layer0_body(x_ref, a_ref, wr_ref, wl_ref, b_ref, o_ref):
    acc = a_ref[...] * wl_ref[...] + x_ref[...] * wr_ref[...] + b_ref[...]
    o_ref[...] = jnp.maximum(acc, 0.0)


def _layer_body(h_ref, a_ref, wr_ref, wl_ref, b_ref, o_ref):
    acc = jnp.dot(a_ref[...], wl_ref[...], preferred_element_type=jnp.float32)
    acc += jnp.dot(h_ref[...], wr_ref[...], preferred_element_type=jnp.float32)
    acc += b_ref[...]
    o_ref[...] = jnp.maximum(acc, 0.0)


def _layer_body2(h_ref, alo_ref, ahi_ref, wr_ref, wllo_ref, wlhi_ref,
                 b_ref, o_ref):
    acc = jnp.dot(alo_ref[...], wllo_ref[...], preferred_element_type=jnp.float32)
    acc += jnp.dot(ahi_ref[...], wlhi_ref[...], preferred_element_type=jnp.float32)
    acc += jnp.dot(h_ref[...], wr_ref[...], preferred_element_type=jnp.float32)
    acc += b_ref[...]
    o_ref[...] = jnp.maximum(acc, 0.0)


def _dense_update2(h, alo, ahi, Wl, bl, Wr):
    n = h.shape[0]
    blk = 2000
    grid = n // blk
    wlt = Wl.T
    return pl.pallas_call(
        _layer_body2,
        grid=(grid,),
        in_specs=[
            pl.BlockSpec((blk, _H), lambda i: (i, 0)),
            pl.BlockSpec((blk, 32), lambda i: (i, 0)),
            pl.BlockSpec((blk, 32), lambda i: (i, 0)),
            pl.BlockSpec((_H, _H), lambda i: (0, 0)),
            pl.BlockSpec((32, _H), lambda i: (0, 0)),
            pl.BlockSpec((32, _H), lambda i: (0, 0)),
            pl.BlockSpec((1, _H), lambda i: (0, 0)),
        ],
        out_specs=pl.BlockSpec((blk, _H), lambda i: (i, 0)),
        out_shape=jax.ShapeDtypeStruct((n, _H), jnp.float32),
    )(h, alo, ahi, Wr.T, wlt[:32], wlt[32:], bl[None, :])


def _dense_update(h, agg, Wl, bl, Wr):
    n, din = h.shape
    blk = 2000
    grid = n // blk
    body = _layer0_body if din == 1 else _layer_body
    return pl.pallas_call(
        body,
        grid=(grid,),
        in_specs=[
            pl.BlockSpec((blk, din), lambda i: (i, 0)),
            pl.BlockSpec((blk, din), lambda i: (i, 0)),
            pl.BlockSpec((din, _H), lambda i: (0, 0)),
            pl.BlockSpec((din, _H), lambda i: (0, 0)),
            pl.BlockSpec((1, _H), lambda i: (0, 0)),
        ],
        out_specs=pl.BlockSpec((blk, _H), lambda i: (i, 0)),
        out_shape=jax.ShapeDtypeStruct((n, _H), jnp.float32),
    )(h, agg, Wr.T, Wl.T, bl[None, :])


def _combine_body(zs_ref, zd_ref, wc_ref, bc_ref, o_ref):
    ef = zs_ref[...] * zd_ref[...]
    s = jnp.dot(ef, wc_ref[...], preferred_element_type=jnp.float32)[:, 0]
    o_ref[...] = jax.nn.sigmoid(s + bc_ref[0, 0])


def _combine(z_src, z_dst, Wc, bc):
    q = z_src.shape[0]
    blk = 8192
    grid = q // blk
    return pl.pallas_call(
        _combine_body,
        grid=(grid,),
        in_specs=[
            pl.BlockSpec((blk, _H), lambda i: (i, 0)),
            pl.BlockSpec((blk, _H), lambda i: (i, 0)),
            pl.BlockSpec((_H, 1), lambda i: (0, 0)),
            pl.BlockSpec((1, 1), lambda i: (0, 0)),
        ],
        out_specs=pl.BlockSpec((blk,), lambda i: (i,)),
        out_shape=jax.ShapeDtypeStruct((q,), jnp.float32),
    )(z_src, z_dst, Wc.T, bc[None, :])


def kernel(x, edge_index, query_edges, Wl0, bl0, Wr0, Wl1, bl1, Wr1,
           Wl2, bl2, Wr2, Wl3, bl3, Wr3, Wc, bc):
    n = x.shape[0]
    e = edge_index.shape[1]
    src, dst = edge_index[0], edge_index[1]
    src_p, _ = _pad_idx(src, _NW * _W)
    e_pad = src_p.shape[0]
    dst_p = jnp.concatenate([dst, jnp.full((e_pad - e,), n, dtype=dst.dtype)])
    sdst, ssrc = lax.sort((dst_p, src_p), num_keys=1)

    offs = jnp.searchsorted(sdst, jnp.arange(33, dtype=jnp.int32) * _NPT,
                            ).astype(jnp.int32)
    offs = jnp.concatenate([offs, jnp.zeros((31,), jnp.int32)])
    tailn = jnp.full((2 * _SW,), n, dtype=jnp.int32)
    tail0 = jnp.zeros((2 * _SW,), jnp.int32)
    sdst_p = jnp.concatenate([sdst, tailn])
    sa = jnp.concatenate([ssrc * 2, tail0])
    sb = jnp.concatenate([ssrc * 2 + 1, tail0])

    nwin = e_pad // (_NW * _W)
    msgs = _sc_gather_call(nwin, 1, x, ssrc)[:, None]
    agg = jax.ops.segment_max(msgs, sdst, num_segments=n,
                              indices_are_sorted=True)
    agg = jnp.where(jnp.isfinite(agg), agg, 0.0)
    h = _dense_update(x, agg, Wl0, bl0, Wr0)

    for (Wl, bl, Wr) in ((Wl1, bl1, Wr1), (Wl2, bl2, Wr2), (Wl3, bl3, Wr3)):
        alo, ahi = _sc_segmax(h.reshape(2 * n, 32), sdst_p, sa, sb, offs, n)
        h = _dense_update2(h, alo, ahi, Wl, bl, Wr)

    q = query_edges.shape[1]
    qs_p, _ = _pad_idx(query_edges[0], _NW * _W)
    qd_p, _ = _pad_idx(query_edges[1], _NW * _W)
    z_src = _sc_gather(h, qs_p)
    z_dst = _sc_gather(h, qd_p)
    return _combine(z_src, z_dst, Wc, bc)[:q]
